# baseline (speedup 1.0000x reference)
import functools

import jax
import jax.numpy as jnp
from jax import lax
from jax.experimental import pallas as pl
from jax.experimental.pallas import tpu as pltpu
from jax.experimental.pallas import tpu_sc as plsc

EMBED = 256
VOCAB = 96
CDIM = 50
WLEN = 21
KW = 5
OUT_LEN = WLEN - KW + 1
DPAD = 128
BLK = 512

NC, NS = 2, 16
NW = NC * NS


def _sc_gather(n_rows):
    per_w = n_rows // NW
    n_chunk = 8
    chunk = per_w // n_chunk
    mesh = plsc.VectorSubcoreMesh(core_axis_name="c", subcore_axis_name="s")

    @functools.partial(
        pl.kernel, mesh=mesh,
        out_type=jax.ShapeDtypeStruct((n_rows, DPAD), jnp.float32),
        scratch_types=[
            pltpu.VMEM((chunk,), jnp.int32),
            pltpu.VMEM((chunk, DPAD), jnp.float32),
            pltpu.SemaphoreType.DMA,
        ],
    )
    def gather_kernel(table_hbm, idx_hbm, out_hbm, idx_v, rows_v, sem):
        wid = lax.axis_index("s") * NC + lax.axis_index("c")
        base = wid * per_w
        for c in range(n_chunk):
            off = base + c * chunk
            pltpu.sync_copy(idx_hbm.at[pl.ds(off, chunk)], idx_v)
            pltpu.async_copy(table_hbm.at[idx_v], rows_v, sem).wait()
            pltpu.sync_copy(rows_v, out_hbm.at[pl.ds(off, chunk)])

    return gather_kernel


def _dense_body(e_ref, w_ref, cb_ref, wp_ref, bp_ref, wg_ref, bg_ref, out_ref):
    e = e_ref[...].astype(jnp.bfloat16)
    conv = jax.lax.dot_general(
        e[0:OUT_LEN].reshape(OUT_LEN * BLK, DPAD), w_ref[0],
        (((1,), (0,)), ((), ())), preferred_element_type=jnp.float32)
    for k in range(1, KW):
        conv += jax.lax.dot_general(
            e[k:k + OUT_LEN].reshape(OUT_LEN * BLK, DPAD), w_ref[k],
            (((1,), (0,)), ((), ())), preferred_element_type=jnp.float32)
    h = jax.nn.relu(
        jnp.max(conv.reshape(OUT_LEN, BLK, EMBED), axis=0) + cb_ref[...])
    proj = jax.nn.relu(
        jax.lax.dot_general(h, wp_ref[...], (((1,), (0,)), ((), ())),
                            preferred_element_type=jnp.float32) + bp_ref[...])
    gate = jax.nn.sigmoid(
        jax.lax.dot_general(h, wg_ref[...], (((1,), (0,)), ((), ())),
                            preferred_element_type=jnp.float32) + bg_ref[...])
    out_ref[...] = gate * proj + (1.0 - gate) * h


def kernel(input_ids, char_emb, conv_w, conv_b, W_proj, b_proj, W_gate, b_gate):
    sent_len, batch, wlen = input_ids.shape
    n = sent_len * batch
    ids_t = input_ids.reshape(n, wlen).astype(jnp.int32).T
    n_rows = wlen * n

    table = jnp.pad(char_emb, ((0, 0), (0, DPAD - CDIM)))

    e = _sc_gather(n_rows)(table, ids_t.reshape(n_rows)).reshape(wlen, n, DPAD)

    w = jnp.transpose(conv_w, (2, 1, 0))
    w = jnp.pad(w, ((0, 0), (0, DPAD - CDIM), (0, 0))).astype(jnp.bfloat16)

    grid = (n // BLK,)
    out = pl.pallas_call(
        _dense_body,
        grid=grid,
        in_specs=[
            pl.BlockSpec((wlen, BLK, DPAD), lambda i: (0, i, 0)),
            pl.BlockSpec((KW, DPAD, EMBED), lambda i: (0, 0, 0)),
            pl.BlockSpec((1, EMBED), lambda i: (0, 0)),
            pl.BlockSpec((EMBED, EMBED), lambda i: (0, 0)),
            pl.BlockSpec((1, EMBED), lambda i: (0, 0)),
            pl.BlockSpec((EMBED, EMBED), lambda i: (0, 0)),
            pl.BlockSpec((1, EMBED), lambda i: (0, 0)),
        ],
        out_specs=pl.BlockSpec((BLK, EMBED), lambda i: (i, 0)),
        out_shape=jax.ShapeDtypeStruct((n, EMBED), jnp.float32),
    )(e, w, conv_b.reshape(1, EMBED), W_proj.T, b_proj.reshape(1, EMBED),
      W_gate.T, b_gate.reshape(1, EMBED))

    return out.reshape(sent_len, batch, EMBED)


# speedup vs baseline: 2.6237x; 2.6237x over previous
import jax
import jax.numpy as jnp
from jax.experimental import pallas as pl

EMBED = 256
VOCAB = 96
CDIM = 50
WLEN = 21
KW = 5
OUT_LEN = WLEN - KW + 1
VPAD = 128
DPAD = 64
BLK = 512


def _fused_body(ids_ref, ce_ref, w_ref, cb_ref, wp_ref, bp_ref, wg_ref,
                bg_ref, out_ref):
    ids = ids_ref[...]
    iota = jax.lax.broadcasted_iota(jnp.int32, (WLEN, BLK, VPAD), 2)
    oh = (iota == ids[:, :, None]).astype(jnp.bfloat16)
    e = jax.lax.dot_general(
        oh.reshape(WLEN * BLK, VPAD), ce_ref[...],
        (((1,), (0,)), ((), ())),
        preferred_element_type=jnp.float32).astype(jnp.bfloat16)
    e = e.reshape(WLEN, BLK, DPAD)
    conv = jax.lax.dot_general(
        e[0:OUT_LEN].reshape(OUT_LEN * BLK, DPAD), w_ref[0],
        (((1,), (0,)), ((), ())), preferred_element_type=jnp.float32)
    for k in range(1, KW):
        conv += jax.lax.dot_general(
            e[k:k + OUT_LEN].reshape(OUT_LEN * BLK, DPAD), w_ref[k],
            (((1,), (0,)), ((), ())), preferred_element_type=jnp.float32)
    h = jax.nn.relu(
        jnp.max(conv.reshape(OUT_LEN, BLK, EMBED), axis=0) + cb_ref[...])
    proj = jax.nn.relu(
        jax.lax.dot_general(h, wp_ref[...], (((1,), (0,)), ((), ())),
                            preferred_element_type=jnp.float32) + bp_ref[...])
    gate = jax.nn.sigmoid(
        jax.lax.dot_general(h, wg_ref[...], (((1,), (0,)), ((), ())),
                            preferred_element_type=jnp.float32) + bg_ref[...])
    out_ref[...] = gate * proj + (1.0 - gate) * h


def kernel(input_ids, char_emb, conv_w, conv_b, W_proj, b_proj, W_gate, b_gate):
    sent_len, batch, wlen = input_ids.shape
    n = sent_len * batch
    ids_t = input_ids.reshape(n, wlen).astype(jnp.int32).T

    ce = jnp.pad(char_emb, ((0, VPAD - VOCAB), (0, DPAD - CDIM)))
    ce = ce.astype(jnp.bfloat16)
    w = jnp.transpose(conv_w, (2, 1, 0))
    w = jnp.pad(w, ((0, 0), (0, DPAD - CDIM), (0, 0))).astype(jnp.bfloat16)

    grid = (n // BLK,)
    out = pl.pallas_call(
        _fused_body,
        grid=grid,
        in_specs=[
            pl.BlockSpec((wlen, BLK), lambda i: (0, i)),
            pl.BlockSpec((VPAD, DPAD), lambda i: (0, 0)),
            pl.BlockSpec((KW, DPAD, EMBED), lambda i: (0, 0, 0)),
            pl.BlockSpec((1, EMBED), lambda i: (0, 0)),
            pl.BlockSpec((EMBED, EMBED), lambda i: (0, 0)),
            pl.BlockSpec((1, EMBED), lambda i: (0, 0)),
            pl.BlockSpec((EMBED, EMBED), lambda i: (0, 0)),
            pl.BlockSpec((1, EMBED), lambda i: (0, 0)),
        ],
        out_specs=pl.BlockSpec((BLK, EMBED), lambda i: (i, 0)),
        out_shape=jax.ShapeDtypeStruct((n, EMBED), jnp.float32),
    )(ids_t, ce, w, conv_b.reshape(1, EMBED), W_proj.T,
      b_proj.reshape(1, EMBED), W_gate.T, b_gate.reshape(1, EMBED))

    return out.reshape(sent_len, batch, EMBED)
